# baseline (speedup 1.0000x reference)
import functools

import jax
import jax.numpy as jnp
from jax import lax
from jax.experimental import pallas as pl
from jax.experimental.pallas import tpu as pltpu
from jax.experimental.pallas import tpu_sc as plsc

NC = 2
NS = 16
L = 16
NW = NC * NS


def kernel(user, item, user_table, item_table, user_bias, item_bias):
    B = user.shape[0]
    D = user_table.shape[1]
    bpw = B // NW

    mesh = plsc.VectorSubcoreMesh(
        core_axis_name="c", subcore_axis_name="s", num_cores=NC, num_subcores=NS
    )

    @functools.partial(
        pl.kernel,
        out_type=jax.ShapeDtypeStruct((B, D), jnp.float32),
        mesh=mesh,
        compiler_params=pltpu.CompilerParams(use_tc_tiling_on_sc=False),
        scratch_types=[
            pltpu.VMEM((bpw,), jnp.int32),
            pltpu.VMEM((bpw,), jnp.int32),
            pltpu.VMEM((bpw, D), jnp.float32),
            pltpu.VMEM((bpw, D), jnp.float32),
            pltpu.VMEM((bpw,), jnp.float32),
            pltpu.VMEM((bpw,), jnp.float32),
            pltpu.VMEM((bpw, D), jnp.float32),
            pltpu.SemaphoreType.DMA,
            pltpu.SemaphoreType.DMA,
            pltpu.SemaphoreType.DMA,
            pltpu.SemaphoreType.DMA,
        ],
    )
    def gmf(user_hbm, item_hbm, utab_hbm, itab_hbm, ubias_hbm, ibias_hbm,
            out_hbm, uidx_v, iidx_v, urows_v, irows_v, ub_v, ib_v, o_v,
            s0, s1, s2, s3):
        wid = lax.axis_index("s") * NC + lax.axis_index("c")
        base = wid * bpw

        pltpu.sync_copy(user_hbm.at[pl.ds(base, bpw)], uidx_v)
        pltpu.sync_copy(item_hbm.at[pl.ds(base, bpw)], iidx_v)

        cu = pltpu.async_copy(utab_hbm.at[uidx_v], urows_v, s0)
        ci = pltpu.async_copy(itab_hbm.at[iidx_v], irows_v, s1)
        cub = pltpu.async_copy(ubias_hbm.at[uidx_v], ub_v, s2)
        cib = pltpu.async_copy(ibias_hbm.at[iidx_v], ib_v, s3)
        cub.wait()
        cib.wait()
        cu.wait()
        ci.wait()

        def blk(bi, _):
            b0 = bi * L
            ub16 = ub_v[pl.ds(b0, L)]
            ib16 = ib_v[pl.ds(b0, L)]
            for j in range(L):
                b = b0 + j
                ubb = jnp.full((L,), ub16[j])
                ibb = jnp.full((L,), ib16[j])
                for q in range(D // L):
                    sl = pl.ds(q * L, L)
                    o_v[b, sl] = (urows_v[b, sl] + ubb) * (irows_v[b, sl] + ibb)
            return 0

        lax.fori_loop(0, bpw // L, blk, 0)

        pltpu.sync_copy(o_v, out_hbm.at[pl.ds(base, bpw)])

    return gmf(user, item, user_table, item_table,
               user_bias.reshape(-1), item_bias.reshape(-1))
